# baseline (speedup 1.0000x reference)
import functools

import jax
import jax.numpy as jnp
from jax import lax
from jax.experimental import pallas as pl
from jax.experimental.pallas import tpu as pltpu
from jax.experimental.pallas import tpu_sc as plsc

NC = 2
NS = 16
NW = NC * NS
LANES = 16
CHUNK = 10240
UNROLL = 4
RB = 1024
TC_STEPS = 35


@functools.lru_cache(maxsize=None)
def _sc_partial_matmul(nnz: int, start: int, ch: int):
    n_edges = nnz - start
    assert n_edges % ch == 0 and start % ch == 0 and ch % 128 == 0
    assert ch % (LANES * UNROLL) == 0
    n_chunks = n_edges // ch
    assert n_chunks >= NW
    mesh = plsc.VectorSubcoreMesh(core_axis_name="c", subcore_axis_name="s")

    @functools.partial(
        pl.kernel,
        mesh=mesh,
        compiler_params=pltpu.CompilerParams(needs_layout_passes=False),
        out_type=jax.ShapeDtypeStruct((NW * 4 * LANES,), jnp.float32),
        scratch_types=[
            pltpu.VMEM((2, ch), jnp.float32),
            pltpu.VMEM((2, ch), jnp.float32),
            pltpu.VMEM((2, ch), jnp.float32),
            pltpu.VMEM((2, ch), jnp.float32),
            pltpu.VMEM((4 * LANES,), jnp.float32),
            pltpu.SemaphoreType.DMA,
            pltpu.SemaphoreType.DMA,
        ],
    )
    def k(t_hbm, w_hbm, out_hbm, t_a, t_b, w_a, w_b, acc_v, sem_a, sem_b):
        wid = lax.axis_index("s") * NC + lax.axis_index("c")
        my_n = (n_chunks - wid + NW - 1) // NW
        zero = jnp.zeros((LANES,), jnp.float32)
        zeros8 = (zero,) * 8

        def start_dma(i, t_buf, w_buf, sem):
            off = start + (wid + i * NW) * ch
            pltpu.async_copy(t_hbm.at[:, pl.ds(off, ch)], t_buf, sem)
            pltpu.async_copy(w_hbm.at[:, pl.ds(off, ch)], w_buf, sem)

        def wait_dma(t_buf, w_buf, sem):
            pltpu.make_async_copy(t_hbm.at[:, pl.ds(0, ch)], t_buf, sem).wait()
            pltpu.make_async_copy(w_hbm.at[:, pl.ds(0, ch)], w_buf, sem).wait()

        def compute(t_buf, w_buf, accs):
            def step(s, accs2):
                (a00, a01, a10, a11, b00, b01, b10, b11) = accs2
                for u in range(UNROLL):
                    base = (s * UNROLL + u) * LANES
                    t0 = t_buf[0, pl.ds(base, LANES)]
                    t1 = t_buf[1, pl.ds(base, LANES)]
                    w0 = w_buf[0, pl.ds(base, LANES)]
                    w1 = w_buf[1, pl.ds(base, LANES)]
                    if u % 2 == 0:
                        a00 += t0 * w0
                        a01 += t0 * w1
                        a10 += t1 * w0
                        a11 += t1 * w1
                    else:
                        b00 += t0 * w0
                        b01 += t0 * w1
                        b10 += t1 * w0
                        b11 += t1 * w1
                return (a00, a01, a10, a11, b00, b01, b10, b11)

            return lax.fori_loop(0, ch // (LANES * UNROLL), step, accs)

        start_dma(0, t_a, w_a, sem_a)

        def pair_body(p, accs):
            i1 = 2 * p + 1

            @pl.when(i1 < my_n)
            def _():
                start_dma(i1, t_b, w_b, sem_b)

            wait_dma(t_a, w_a, sem_a)
            accs = compute(t_a, w_a, accs)

            @pl.when(i1 + 1 < my_n)
            def _():
                start_dma(i1 + 1, t_a, w_a, sem_a)

            def odd(accs2):
                wait_dma(t_b, w_b, sem_b)
                return compute(t_b, w_b, accs2)

            return lax.cond(i1 < my_n, odd, lambda a: a, accs)

        accs = lax.fori_loop(0, (my_n + 1) // 2, pair_body, zeros8)
        (a00, a01, a10, a11, b00, b01, b10, b11) = accs
        acc_v[pl.ds(0, LANES)] = a00 + b00
        acc_v[pl.ds(LANES, LANES)] = a01 + b01
        acc_v[pl.ds(2 * LANES, LANES)] = a10 + b10
        acc_v[pl.ds(3 * LANES, LANES)] = a11 + b11
        pltpu.sync_copy(acc_v, out_hbm.at[pl.ds(wid * 4 * LANES, 4 * LANES)])

    return k


def _tc_block(t_ref, w_ref, out_ref):
    @pl.when(pl.program_id(0) == 0)
    def _():
        out_ref[...] = jnp.zeros_like(out_ref)

    even = (lax.broadcasted_iota(jnp.int32, (8, 128), 0) % 2) == 0
    zeros = jnp.zeros((8, 128), jnp.float32)
    tc_unroll = 8
    n_acc = 4

    def body(j, accs):
        ads = list(accs[:n_acc])
        acs = list(accs[n_acc:])
        for u in range(tc_unroll):
            base = (j * tc_unroll + u) * 8
            tj = t_ref[pl.ds(base, 8), :]
            wj = w_ref[pl.ds(base, 8), :]
            w_up = pltpu.roll(wj, 7, 0)
            w_dn = pltpu.roll(wj, 1, 0)
            ws = jnp.where(even, w_up, w_dn)
            ads[u % n_acc] += tj * wj
            acs[u % n_acc] += tj * ws
        return tuple(ads) + tuple(acs)

    accs = lax.fori_loop(
        0, RB // (8 * tc_unroll), body, (zeros,) * (2 * n_acc))
    out_ref[0] += sum(accs[:n_acc], zeros)
    out_ref[1] += sum(accs[n_acc:], zeros)


@functools.lru_cache(maxsize=None)
def _tc_partial_matmul(n_steps: int):
    return pl.pallas_call(
        _tc_block,
        grid=(n_steps,),
        in_specs=[
            pl.BlockSpec((RB, 128), lambda i: (i, 0)),
            pl.BlockSpec((RB, 128), lambda i: (i, 0)),
        ],
        out_specs=pl.BlockSpec((2, 8, 128), lambda i: (0, 0, 0)),
        out_shape=jax.ShapeDtypeStruct((2, 8, 128), jnp.float32),
    )


def kernel(temp2, weight, bias):
    nnz = temp2.shape[1]
    nb = nnz // 128
    wt = weight.T
    vt = temp2.reshape(2, nb, 128).transpose(1, 0, 2).reshape(2 * nb, 128)
    vw = wt.reshape(2, nb, 128).transpose(1, 0, 2).reshape(2 * nb, 128)

    tc_edges = RB * 64 * TC_STEPS
    partials = _sc_partial_matmul(nnz, tc_edges, CHUNK)(temp2, wt)
    tc_acc = _tc_partial_matmul(TC_STEPS)(vt, vw)

    s = jnp.sum(partials.reshape(NW, 4, LANES), axis=(0, 2))
    pd, pc = tc_acc[0], tc_acc[1]
    t00 = jnp.sum(pd[0::2])
    t11 = jnp.sum(pd[1::2])
    t01 = jnp.sum(pc[0::2])
    t10 = jnp.sum(pc[1::2])
    tc_mat = jnp.stack([jnp.stack([t00, t01]), jnp.stack([t10, t11])])
    x = s.reshape(2, 2) + tc_mat + bias
    return jax.nn.relu(x)


# speedup vs baseline: 1.0412x; 1.0412x over previous
import functools

import jax
import jax.numpy as jnp
from jax import lax
from jax.experimental import pallas as pl
from jax.experimental.pallas import tpu as pltpu
from jax.experimental.pallas import tpu_sc as plsc

NC = 2
NS = 16
NW = NC * NS
LANES = 16
CHUNK = 5120
UNROLL = 4
RB = 1024
TC_STEPS = 35


@functools.lru_cache(maxsize=None)
def _sc_partial_matmul(nnz: int, start: int, ch: int):
    n_edges = nnz - start
    assert n_edges % ch == 0 and start % ch == 0 and ch % 128 == 0
    assert ch % (LANES * UNROLL) == 0
    n_chunks = n_edges // ch
    assert n_chunks >= NW
    mesh = plsc.VectorSubcoreMesh(core_axis_name="c", subcore_axis_name="s")

    @functools.partial(
        pl.kernel,
        mesh=mesh,
        compiler_params=pltpu.CompilerParams(needs_layout_passes=False),
        out_type=jax.ShapeDtypeStruct((NW * 4 * LANES,), jnp.float32),
        scratch_types=[
            pltpu.VMEM((2, ch), jnp.float32),
            pltpu.VMEM((2, ch), jnp.float32),
            pltpu.VMEM((2, ch), jnp.float32),
            pltpu.VMEM((2, ch), jnp.float32),
            pltpu.VMEM((4 * LANES,), jnp.float32),
            pltpu.SemaphoreType.DMA,
            pltpu.SemaphoreType.DMA,
        ],
    )
    def k(t_hbm, w_hbm, out_hbm, t_a, t_b, w_a, w_b, acc_v, sem_a, sem_b):
        wid = lax.axis_index("s") * NC + lax.axis_index("c")
        my_n = (n_chunks - wid + NW - 1) // NW
        zero = jnp.zeros((LANES,), jnp.float32)
        zeros8 = (zero,) * 8

        def start_dma(i, t_buf, w_buf, sem):
            off = start + (wid + i * NW) * ch
            pltpu.async_copy(t_hbm.at[:, pl.ds(off, ch)], t_buf, sem)
            pltpu.async_copy(w_hbm.at[:, pl.ds(off, ch)], w_buf, sem)

        def wait_dma(t_buf, w_buf, sem):
            pltpu.make_async_copy(t_hbm.at[:, pl.ds(0, ch)], t_buf, sem).wait()
            pltpu.make_async_copy(w_hbm.at[:, pl.ds(0, ch)], w_buf, sem).wait()

        def compute(t_buf, w_buf, accs):
            def step(s, accs2):
                (a00, a01, a10, a11, b00, b01, b10, b11) = accs2
                for u in range(UNROLL):
                    base = (s * UNROLL + u) * LANES
                    t0 = t_buf[0, pl.ds(base, LANES)]
                    t1 = t_buf[1, pl.ds(base, LANES)]
                    w0 = w_buf[0, pl.ds(base, LANES)]
                    w1 = w_buf[1, pl.ds(base, LANES)]
                    if u % 2 == 0:
                        a00 += t0 * w0
                        a01 += t0 * w1
                        a10 += t1 * w0
                        a11 += t1 * w1
                    else:
                        b00 += t0 * w0
                        b01 += t0 * w1
                        b10 += t1 * w0
                        b11 += t1 * w1
                return (a00, a01, a10, a11, b00, b01, b10, b11)

            return lax.fori_loop(0, ch // (LANES * UNROLL), step, accs)

        start_dma(0, t_a, w_a, sem_a)

        def pair_body(p, accs):
            i1 = 2 * p + 1

            @pl.when(i1 < my_n)
            def _():
                start_dma(i1, t_b, w_b, sem_b)

            wait_dma(t_a, w_a, sem_a)
            accs = compute(t_a, w_a, accs)

            @pl.when(i1 + 1 < my_n)
            def _():
                start_dma(i1 + 1, t_a, w_a, sem_a)

            def odd(accs2):
                wait_dma(t_b, w_b, sem_b)
                return compute(t_b, w_b, accs2)

            return lax.cond(i1 < my_n, odd, lambda a: a, accs)

        accs = lax.fori_loop(0, (my_n + 1) // 2, pair_body, zeros8)
        (a00, a01, a10, a11, b00, b01, b10, b11) = accs
        acc_v[pl.ds(0, LANES)] = a00 + b00
        acc_v[pl.ds(LANES, LANES)] = a01 + b01
        acc_v[pl.ds(2 * LANES, LANES)] = a10 + b10
        acc_v[pl.ds(3 * LANES, LANES)] = a11 + b11
        pltpu.sync_copy(acc_v, out_hbm.at[pl.ds(wid * 4 * LANES, 4 * LANES)])

    return k


def _tc_block(t_ref, w_ref, out_ref):
    @pl.when(pl.program_id(0) == 0)
    def _():
        out_ref[...] = jnp.zeros_like(out_ref)

    even = (lax.broadcasted_iota(jnp.int32, (8, 128), 0) % 2) == 0
    zeros = jnp.zeros((8, 128), jnp.float32)
    tc_unroll = 8
    n_acc = 4

    def body(j, accs):
        ads = list(accs[:n_acc])
        acs = list(accs[n_acc:])
        for u in range(tc_unroll):
            base = (j * tc_unroll + u) * 8
            tj = t_ref[pl.ds(base, 8), :]
            wj = w_ref[pl.ds(base, 8), :]
            w_up = pltpu.roll(wj, 7, 0)
            w_dn = pltpu.roll(wj, 1, 0)
            ws = jnp.where(even, w_up, w_dn)
            ads[u % n_acc] += tj * wj
            acs[u % n_acc] += tj * ws
        return tuple(ads) + tuple(acs)

    accs = lax.fori_loop(
        0, RB // (8 * tc_unroll), body, (zeros,) * (2 * n_acc))
    out_ref[0] += sum(accs[:n_acc], zeros)
    out_ref[1] += sum(accs[n_acc:], zeros)


@functools.lru_cache(maxsize=None)
def _tc_partial_matmul(n_steps: int):
    return pl.pallas_call(
        _tc_block,
        grid=(n_steps,),
        in_specs=[
            pl.BlockSpec((RB, 128), lambda i: (i, 0)),
            pl.BlockSpec((RB, 128), lambda i: (i, 0)),
        ],
        out_specs=pl.BlockSpec((2, 8, 128), lambda i: (0, 0, 0)),
        out_shape=jax.ShapeDtypeStruct((2, 8, 128), jnp.float32),
    )


def kernel(temp2, weight, bias):
    nnz = temp2.shape[1]
    nb = nnz // 128
    wt = weight.T
    vt = temp2.reshape(2, nb, 128).transpose(1, 0, 2).reshape(2 * nb, 128)
    vw = wt.reshape(2, nb, 128).transpose(1, 0, 2).reshape(2 * nb, 128)

    tc_edges = RB * 64 * TC_STEPS
    partials = _sc_partial_matmul(nnz, tc_edges, CHUNK)(temp2, wt)
    tc_acc = _tc_partial_matmul(TC_STEPS)(vt, vw)

    s = jnp.sum(partials.reshape(NW, 4, LANES), axis=(0, 2))
    pd, pc = tc_acc[0], tc_acc[1]
    t00 = jnp.sum(pd[0::2])
    t11 = jnp.sum(pd[1::2])
    t01 = jnp.sum(pc[0::2])
    t10 = jnp.sum(pc[1::2])
    tc_mat = jnp.stack([jnp.stack([t00, t01]), jnp.stack([t10, t11])])
    x = s.reshape(2, 2) + tc_mat + bias
    return jax.nn.relu(x)
